# baseline (speedup 1.0000x reference)
import functools

import jax
import jax.numpy as jnp
from jax import lax
from jax.experimental import pallas as pl
from jax.experimental.pallas import tpu as pltpu
from jax.experimental.pallas import tpu_sc as plsc

B, N, DIM = 16, 1024, 256
BN = B * N
K = 1024
M = 1024
S = 4
BNS = BN // S
NBS = BNS // M

NC, NS = 2, 16
NW = NC * NS
CH = BNS // NW


def _make_tc_body(s, has_in):
    def body(*args):
        if has_in:
            x_ref, e_ref, _dist_in, dist_out, idx_ref, esq, dvm, sem0, sem1 = args
        else:
            x_ref, e_ref, dist_out, idx_ref, esq, dvm, sem0, sem1 = args
        i = pl.program_id(0)
        slot = lax.rem(i, 2)
        base = s * NBS

        @pl.when(i == 0)
        def _():
            e0 = e_ref[...]
            esq[...] = jnp.sum(e0 * e0, axis=1)[None, :]

        @pl.when(jnp.logical_and(i >= 2, slot == 0))
        def _():
            pltpu.make_async_copy(
                dvm.at[0], dist_out.at[pl.ds((base + i - 2) * M, M), :], sem0
            ).wait()

        @pl.when(jnp.logical_and(i >= 2, slot == 1))
        def _():
            pltpu.make_async_copy(
                dvm.at[1], dist_out.at[pl.ds((base + i - 2) * M, M), :], sem1
            ).wait()

        x = x_ref[...]
        e = e_ref[...]
        cross = jax.lax.dot_general(
            x, e, (((1,), (1,)), ((), ())), preferred_element_type=jnp.float32
        )
        x_sq = jnp.sum(x * x, axis=1, keepdims=True)
        dist = x_sq + esq[...] - 2.0 * cross
        dvm[slot] = dist

        @pl.when(slot == 0)
        def _():
            pltpu.make_async_copy(
                dvm.at[0], dist_out.at[pl.ds((base + i) * M, M), :], sem0
            ).start()

        @pl.when(slot == 1)
        def _():
            pltpu.make_async_copy(
                dvm.at[1], dist_out.at[pl.ds((base + i) * M, M), :], sem1
            ).start()

        m = jnp.min(dist, axis=1, keepdims=True)
        kiota = jax.lax.broadcasted_iota(jnp.int32, (M, K), 1)
        masked = jnp.where(dist == m, kiota, K)
        idx = jnp.min(masked, axis=1).astype(jnp.int32)
        idx_ref[...] = idx.reshape(1, 8, 128)

        @pl.when(i == NBS - 1)
        def _():
            pltpu.make_async_copy(
                dvm.at[(NBS - 2) % 2],
                dist_out.at[pl.ds((base + NBS - 2) * M, M), :],
                (sem0, sem1)[(NBS - 2) % 2],
            ).wait()
            pltpu.make_async_copy(
                dvm.at[(NBS - 1) % 2],
                dist_out.at[pl.ds((base + NBS - 1) * M, M), :],
                (sem0, sem1)[(NBS - 1) % 2],
            ).wait()

    return body


def _tc_strip(s, xf, e, dist_prev):
    has_in = dist_prev is not None
    in_specs = [
        pl.BlockSpec((M, DIM), lambda i, s=s: (s * NBS + i, 0)),
        pl.BlockSpec((K, DIM), lambda i: (0, 0)),
    ]
    operands = [xf, e]
    kwargs = {}
    if has_in:
        in_specs.append(pl.BlockSpec(memory_space=pl.ANY))
        operands.append(dist_prev)
        kwargs["input_output_aliases"] = {2: 0}
    return pl.pallas_call(
        _make_tc_body(s, has_in),
        grid=(NBS,),
        in_specs=in_specs,
        out_specs=[
            pl.BlockSpec(memory_space=pl.ANY),
            pl.BlockSpec((1, 8, 128), lambda i: (i, 0, 0)),
        ],
        out_shape=[
            jax.ShapeDtypeStruct((BN, K), jnp.float32),
            jax.ShapeDtypeStruct((NBS, 8, 128), jnp.int32),
        ],
        scratch_shapes=[
            pltpu.VMEM((1, K), jnp.float32),
            pltpu.VMEM((2, M, K), jnp.float32),
            pltpu.SemaphoreType.DMA,
            pltpu.SemaphoreType.DMA,
        ],
        **kwargs,
    )(*operands)


@functools.partial(
    pl.kernel,
    mesh=plsc.VectorSubcoreMesh(core_axis_name="c", subcore_axis_name="s"),
    out_type=jax.ShapeDtypeStruct((BNS, DIM), jnp.float32),
    scratch_types=[
        pltpu.VMEM((CH,), jnp.int32),
        pltpu.VMEM((CH, DIM), jnp.float32),
        pltpu.SemaphoreType.DMA,
    ],
)
def _sc_gather_strip(idx_hbm, e_hbm, out_hbm, idx_v, rows_v, sem):
    wid = lax.axis_index("s") * NC + lax.axis_index("c")
    base = wid * CH
    pltpu.sync_copy(idx_hbm.at[pl.ds(base, CH)], idx_v)
    pltpu.async_copy(e_hbm.at[idx_v], rows_v, sem).wait()
    pltpu.sync_copy(rows_v, out_hbm.at[pl.ds(base, CH)])


def kernel(x, embed):
    xf = x.reshape(BN, DIM)
    e = embed.reshape(K, DIM)
    dist = None
    idx_strips = []
    q_strips = []
    for s in range(S):
        dist, idx3 = _tc_strip(s, xf, e, dist)
        idx_s = idx3.reshape(BNS)
        idx_strips.append(idx_s)
        q_strips.append(_sc_gather_strip(idx_s, e))
    idx = jnp.concatenate(idx_strips)
    q = jnp.concatenate(q_strips, axis=0)
    return q.reshape(BN, 1, DIM), idx, dist


# speedup vs baseline: 1.1494x; 1.1494x over previous
import functools

import jax
import jax.numpy as jnp
from jax import lax
from jax.experimental import pallas as pl
from jax.experimental.pallas import tpu as pltpu
from jax.experimental.pallas import tpu_sc as plsc

B, N, DIM = 16, 1024, 256
BN = B * N
K = 1024
M = 1024
NB = BN // M

NC, NS = 2, 16
NW = NC * NS
BPW = BN // NW
CH = 128
NCH = BPW // CH


def _idx_body(x_ref, e_ref, idx_ref, esq_ref):
    @pl.when(pl.program_id(0) == 0)
    def _():
        e0 = e_ref[...]
        esq_ref[...] = jnp.sum(e0 * e0, axis=1)[None, :]

    x = x_ref[...]
    e = e_ref[...]
    cross = jax.lax.dot_general(
        x, e, (((1,), (1,)), ((), ())), preferred_element_type=jnp.float32
    )
    x_sq = jnp.sum(x * x, axis=1, keepdims=True)
    dist = x_sq + esq_ref[...] - 2.0 * cross
    m = jnp.min(dist, axis=1, keepdims=True)
    kiota = jax.lax.broadcasted_iota(jnp.int32, (M, K), 1)
    masked = jnp.where(dist == m, kiota, K)
    idx = jnp.min(masked, axis=1).astype(jnp.int32)
    idx_ref[...] = idx.reshape(1, 8, 128)


def _dist_body(x_ref, e_ref, dist_ref, esq_ref):
    @pl.when(pl.program_id(0) == 0)
    def _():
        e0 = e_ref[...]
        esq_ref[...] = jnp.sum(e0 * e0, axis=1)[None, :]

    x = x_ref[...]
    e = e_ref[...]
    cross = jax.lax.dot_general(
        x, e, (((1,), (1,)), ((), ())), preferred_element_type=jnp.float32
    )
    x_sq = jnp.sum(x * x, axis=1, keepdims=True)
    dist_ref[...] = x_sq + esq_ref[...] - 2.0 * cross


@functools.partial(
    pl.kernel,
    mesh=plsc.VectorSubcoreMesh(core_axis_name="c", subcore_axis_name="s"),
    out_type=jax.ShapeDtypeStruct((BN, DIM), jnp.float32),
    scratch_types=[
        pltpu.VMEM((BPW,), jnp.int32),
        pltpu.VMEM((CH, DIM), jnp.float32),
        pltpu.VMEM((CH, DIM), jnp.float32),
        pltpu.SemaphoreType.DMA,
        pltpu.SemaphoreType.DMA,
        pltpu.SemaphoreType.DMA,
        pltpu.SemaphoreType.DMA,
    ],
)
def _sc_gather(idx_hbm, e_hbm, out_hbm, idx_v, r0, r1, g0, g1, w0, w1):
    wid = lax.axis_index("s") * NC + lax.axis_index("c")
    base = wid * BPW
    pltpu.sync_copy(idx_hbm.at[pl.ds(base, BPW)], idx_v)
    rows = (r0, r1)
    gsem = (g0, g1)
    wsem = (w0, w1)
    gathers = [None] * NCH
    writes = [None] * NCH
    gathers[0] = pltpu.async_copy(
        e_hbm.at[idx_v.at[pl.ds(0, CH)]], rows[0], gsem[0]
    )
    for c in range(NCH):
        if c + 1 < NCH:
            if c - 1 >= 0:
                writes[c - 1].wait()
            gathers[c + 1] = pltpu.async_copy(
                e_hbm.at[idx_v.at[pl.ds((c + 1) * CH, CH)]],
                rows[(c + 1) % 2],
                gsem[(c + 1) % 2],
            )
        gathers[c].wait()
        writes[c] = pltpu.async_copy(
            rows[c % 2], out_hbm.at[pl.ds(base + c * CH, CH)], wsem[c % 2]
        )
    writes[NCH - 2].wait()
    writes[NCH - 1].wait()


def kernel(x, embed):
    xf = x.reshape(BN, DIM)
    e = embed.reshape(K, DIM)
    idx3 = pl.pallas_call(
        _idx_body,
        grid=(NB,),
        in_specs=[
            pl.BlockSpec((M, DIM), lambda i: (i, 0)),
            pl.BlockSpec((K, DIM), lambda i: (0, 0)),
        ],
        out_specs=pl.BlockSpec((1, 8, 128), lambda i: (i, 0, 0)),
        out_shape=jax.ShapeDtypeStruct((NB, 8, 128), jnp.int32),
        scratch_shapes=[pltpu.VMEM((1, K), jnp.float32)],
    )(xf, e)
    idx = idx3.reshape(BN)
    q = _sc_gather(idx, e)
    dist = pl.pallas_call(
        _dist_body,
        grid=(NB,),
        in_specs=[
            pl.BlockSpec((M, DIM), lambda i: (i, 0)),
            pl.BlockSpec((K, DIM), lambda i: (0, 0)),
        ],
        out_specs=pl.BlockSpec((M, K), lambda i: (i, 0)),
        out_shape=jax.ShapeDtypeStruct((BN, K), jnp.float32),
        scratch_shapes=[pltpu.VMEM((1, K), jnp.float32)],
    )(xf, e)
    return q.reshape(BN, 1, DIM), idx, dist


# speedup vs baseline: 1.4254x; 1.2401x over previous
import functools

import jax
import jax.numpy as jnp
from jax import lax
from jax.experimental import pallas as pl
from jax.experimental.pallas import tpu as pltpu
from jax.experimental.pallas import tpu_sc as plsc

B, N, DIM = 16, 1024, 256
BN = B * N
K = 1024
M = 1024
NB = BN // M

NC, NS = 2, 16
NW = NC * NS
BPW = BN // NW
CH = 128
NCH = BPW // CH


def _tc_body(x_ref, e_ref, dist_ref, idx_ref, esq_ref):
    @pl.when(pl.program_id(0) == 0)
    def _():
        e0 = e_ref[...]
        esq_ref[...] = jnp.sum(e0 * e0, axis=1)[None, :]

    x = x_ref[...]
    e = e_ref[...]
    cross = jax.lax.dot_general(
        x, e, (((1,), (1,)), ((), ())), preferred_element_type=jnp.float32
    )
    x_sq = jnp.sum(x * x, axis=1, keepdims=True)
    dist = x_sq + esq_ref[...] - 2.0 * cross
    dist_ref[...] = dist
    m = jnp.min(dist, axis=1, keepdims=True)
    kiota = jax.lax.broadcasted_iota(jnp.int32, (M, K), 1)
    masked = jnp.where(dist == m, kiota, K)
    idx = jnp.min(masked, axis=1).astype(jnp.int32)
    idx_ref[...] = idx.reshape(1, 8, 128)


@functools.partial(
    pl.kernel,
    mesh=plsc.VectorSubcoreMesh(core_axis_name="c", subcore_axis_name="s"),
    out_type=jax.ShapeDtypeStruct((BN, DIM), jnp.float32),
    scratch_types=[
        pltpu.VMEM((BPW,), jnp.int32),
        pltpu.VMEM((CH, DIM), jnp.float32),
        pltpu.VMEM((CH, DIM), jnp.float32),
        pltpu.VMEM((CH, DIM), jnp.float32),
        pltpu.SemaphoreType.DMA,
        pltpu.SemaphoreType.DMA,
        pltpu.SemaphoreType.DMA,
        pltpu.SemaphoreType.DMA,
        pltpu.SemaphoreType.DMA,
        pltpu.SemaphoreType.DMA,
    ],
)
def _sc_gather(idx_hbm, e_hbm, out_hbm, idx_v, r0, r1, r2, g0, g1, g2, w0, w1, w2):
    wid = lax.axis_index("s") * NC + lax.axis_index("c")
    base = wid * BPW
    pltpu.sync_copy(idx_hbm.at[pl.ds(base, BPW)], idx_v)
    rows = (r0, r1, r2)
    gsem = (g0, g1, g2)
    wsem = (w0, w1, w2)
    gathers = [None] * NCH
    writes = [None] * NCH

    def start_gather(c):
        gathers[c] = pltpu.async_copy(
            e_hbm.at[idx_v.at[pl.ds(c * CH, CH)]], rows[c % 3], gsem[c % 3]
        )

    start_gather(0)
    if NCH > 1:
        start_gather(1)
    waited = set()
    for c in range(NCH):
        if c + 2 < NCH:
            if c - 1 >= 0:
                writes[c - 1].wait()
                waited.add(c - 1)
            start_gather(c + 2)
        gathers[c].wait()
        writes[c] = pltpu.async_copy(
            rows[c % 3], out_hbm.at[pl.ds(base + c * CH, CH)], wsem[c % 3]
        )
    for c in range(NCH):
        if c not in waited:
            writes[c].wait()


def kernel(x, embed):
    xf = x.reshape(BN, DIM)
    e = embed.reshape(K, DIM)
    dist, idx3 = pl.pallas_call(
        _tc_body,
        grid=(NB,),
        in_specs=[
            pl.BlockSpec((M, DIM), lambda i: (i, 0)),
            pl.BlockSpec((K, DIM), lambda i: (0, 0)),
        ],
        out_specs=[
            pl.BlockSpec((M, K), lambda i: (i, 0)),
            pl.BlockSpec((1, 8, 128), lambda i: (i, 0, 0)),
        ],
        out_shape=[
            jax.ShapeDtypeStruct((BN, K), jnp.float32),
            jax.ShapeDtypeStruct((NB, 8, 128), jnp.int32),
        ],
        scratch_shapes=[pltpu.VMEM((1, K), jnp.float32)],
    )(xf, e)
    idx = idx3.reshape(BN)
    q = _sc_gather(idx, e)
    return q.reshape(BN, 1, DIM), idx, dist


# speedup vs baseline: 1.5105x; 1.0597x over previous
import functools

import jax
import jax.numpy as jnp
from jax import lax
from jax.experimental import pallas as pl
from jax.experimental.pallas import tpu as pltpu
from jax.experimental.pallas import tpu_sc as plsc

B, N, DIM = 16, 1024, 256
BN = B * N
K = 1024
M = 2048
NB = BN // M

NC, NS = 2, 16
NW = NC * NS
BPW = BN // NW
CH = 128
NCH = BPW // CH


def _tc_body(x_ref, e_ref, dist_ref, idx_ref, esq_ref):
    @pl.when(pl.program_id(0) == 0)
    def _():
        e0 = e_ref[...]
        esq_ref[...] = jnp.sum(e0 * e0, axis=1)[None, :]

    x = x_ref[...]
    e = e_ref[...]
    cross = jax.lax.dot_general(
        x, e, (((1,), (1,)), ((), ())), preferred_element_type=jnp.float32
    )
    x_sq = jnp.sum(x * x, axis=1, keepdims=True)
    dist = x_sq + esq_ref[...] - 2.0 * cross
    dist_ref[...] = dist
    m = jnp.min(dist, axis=1, keepdims=True)
    kiota = jax.lax.broadcasted_iota(jnp.int32, (M, K), 1)
    masked = jnp.where(dist == m, kiota, K)
    idx = jnp.min(masked, axis=1).astype(jnp.int32)
    idx_ref[...] = idx.reshape(1, 16, 128)


@functools.partial(
    pl.kernel,
    mesh=plsc.VectorSubcoreMesh(core_axis_name="c", subcore_axis_name="s"),
    out_type=jax.ShapeDtypeStruct((BN, DIM), jnp.float32),
    scratch_types=[
        pltpu.VMEM((BPW,), jnp.int32),
        pltpu.VMEM((CH, DIM), jnp.float32),
        pltpu.VMEM((CH, DIM), jnp.float32),
        pltpu.VMEM((CH, DIM), jnp.float32),
        pltpu.SemaphoreType.DMA,
        pltpu.SemaphoreType.DMA,
        pltpu.SemaphoreType.DMA,
        pltpu.SemaphoreType.DMA,
        pltpu.SemaphoreType.DMA,
        pltpu.SemaphoreType.DMA,
    ],
)
def _sc_gather(idx_hbm, e_hbm, out_hbm, idx_v, r0, r1, r2, g0, g1, g2, w0, w1, w2):
    wid = lax.axis_index("s") * NC + lax.axis_index("c")
    base = wid * BPW
    pltpu.sync_copy(idx_hbm.at[pl.ds(base, BPW)], idx_v)
    rows = (r0, r1, r2)
    gsem = (g0, g1, g2)
    wsem = (w0, w1, w2)
    gathers = [None] * NCH
    writes = [None] * NCH

    def start_gather(c):
        gathers[c] = pltpu.async_copy(
            e_hbm.at[idx_v.at[pl.ds(c * CH, CH)]], rows[c % 3], gsem[c % 3]
        )

    start_gather(0)
    if NCH > 1:
        start_gather(1)
    waited = set()
    for c in range(NCH):
        if c + 2 < NCH:
            if c - 1 >= 0:
                writes[c - 1].wait()
                waited.add(c - 1)
            start_gather(c + 2)
        gathers[c].wait()
        writes[c] = pltpu.async_copy(
            rows[c % 3], out_hbm.at[pl.ds(base + c * CH, CH)], wsem[c % 3]
        )
    for c in range(NCH):
        if c not in waited:
            writes[c].wait()


def kernel(x, embed):
    xf = x.reshape(BN, DIM)
    e = embed.reshape(K, DIM)
    dist, idx3 = pl.pallas_call(
        _tc_body,
        grid=(NB,),
        in_specs=[
            pl.BlockSpec((M, DIM), lambda i: (i, 0)),
            pl.BlockSpec((K, DIM), lambda i: (0, 0)),
        ],
        out_specs=[
            pl.BlockSpec((M, K), lambda i: (i, 0)),
            pl.BlockSpec((1, 16, 128), lambda i: (i, 0, 0)),
        ],
        out_shape=[
            jax.ShapeDtypeStruct((BN, K), jnp.float32),
            jax.ShapeDtypeStruct((NB, 16, 128), jnp.int32),
        ],
        scratch_shapes=[pltpu.VMEM((1, K), jnp.float32)],
    )(xf, e)
    idx = idx3.reshape(BN)
    q = _sc_gather(idx, e)
    return q.reshape(BN, 1, DIM), idx, dist
